# Initial kernel scaffold; baseline (speedup 1.0000x reference)
import jax
import jax.numpy as jnp
from jax.experimental import pallas as pl


def kernel(x, check_tensor):
    raise NotImplementedError("write your pallas kernel here")


# speedup vs baseline: 2.6613x; 2.6613x over previous
import jax
import jax.numpy as jnp
from jax import lax
from jax.experimental import pallas as pl
from jax.experimental.pallas import tpu as pltpu
from jax.experimental.pallas import tpu_sc as plsc

_NUM_CORES = 2
_NUM_SUBCORES = 16
_NUM_WORKERS = _NUM_CORES * _NUM_SUBCORES
_LANES = 16


def _body(x_hbm, check_hbm, out_hbm, check_v, table_v, x_v):
    n = x_hbm.shape[0]
    c = check_hbm.shape[0]
    per_w = n // _NUM_WORKERS
    wid = lax.axis_index("s") * _NUM_CORES + lax.axis_index("c")
    base = wid * per_w

    pltpu.sync_copy(check_hbm, check_v)
    for j0 in range(c // _LANES):
        table_v[pl.ds(j0 * _LANES, _LANES)] = jnp.zeros((_LANES,), jnp.int32)
    for j0 in reversed(range(c // _LANES)):
        vals = check_v[pl.ds(j0 * _LANES, _LANES)]
        js = lax.iota(jnp.int32, _LANES) + (j0 * _LANES)
        plsc.store_scatter(table_v, [vals], js)

    pltpu.sync_copy(x_hbm.at[pl.ds(base, per_w)], x_v)

    def step(i, carry):
        sl = pl.ds(i * _LANES, _LANES)
        x_v[sl] = plsc.load_gather(table_v, [x_v[sl]])
        return carry

    lax.fori_loop(0, per_w // _LANES, step, 0)
    pltpu.sync_copy(x_v, out_hbm.at[pl.ds(base, per_w)])


def kernel(x, check_tensor):
    n = x.shape[0]
    per_w = n // _NUM_WORKERS
    mesh = plsc.VectorSubcoreMesh(
        core_axis_name="c",
        subcore_axis_name="s",
        num_cores=_NUM_CORES,
        num_subcores=_NUM_SUBCORES,
    )
    f = pl.kernel(
        _body,
        out_type=jax.ShapeDtypeStruct((n,), jnp.int32),
        mesh=mesh,
        scratch_types=[
            pltpu.VMEM((check_tensor.shape[0],), jnp.int32),
            pltpu.VMEM((check_tensor.shape[0],), jnp.int32),
            pltpu.VMEM((per_w,), jnp.int32),
        ],
        compiler_params=pltpu.CompilerParams(needs_layout_passes=False),
    )
    return f(x, check_tensor)


# speedup vs baseline: 4.4128x; 1.6581x over previous
import jax
import jax.numpy as jnp
from jax import lax
from jax.experimental import pallas as pl
from jax.experimental.pallas import tpu as pltpu
from jax.experimental.pallas import tpu_sc as plsc

_NUM_CORES = 2
_NUM_SUBCORES = 16
_NUM_WORKERS = _NUM_CORES * _NUM_SUBCORES
_LANES = 16


def _body(x_hbm, check_hbm, out_hbm, check_v, table_v, x_v, out_v):
    n = x_hbm.shape[0]
    c = check_hbm.shape[0]
    per_w = n // _NUM_WORKERS
    wid = lax.axis_index("s") * _NUM_CORES + lax.axis_index("c")
    base = wid * per_w

    pltpu.sync_copy(check_hbm, check_v)
    for j0 in range(c // _LANES):
        table_v[pl.ds(j0 * _LANES, _LANES)] = jnp.zeros((_LANES,), jnp.int32)
    for j0 in reversed(range(c // _LANES)):
        vals = check_v[pl.ds(j0 * _LANES, _LANES)]
        js = lax.iota(jnp.int32, _LANES) + (j0 * _LANES)
        plsc.store_scatter(table_v, [vals], js)

    pltpu.sync_copy(x_hbm.at[pl.ds(base, per_w)], x_v)

    @plsc.parallel_loop(0, per_w // _LANES, unroll=8)
    def _(i):
        sl = pl.ds(i * _LANES, _LANES)
        out_v[sl] = plsc.load_gather(table_v, [x_v[sl]])

    pltpu.sync_copy(out_v, out_hbm.at[pl.ds(base, per_w)])


def kernel(x, check_tensor):
    n = x.shape[0]
    per_w = n // _NUM_WORKERS
    mesh = plsc.VectorSubcoreMesh(
        core_axis_name="c",
        subcore_axis_name="s",
        num_cores=_NUM_CORES,
        num_subcores=_NUM_SUBCORES,
    )
    f = pl.kernel(
        _body,
        out_type=jax.ShapeDtypeStruct((n,), jnp.int32),
        mesh=mesh,
        scratch_types=[
            pltpu.VMEM((check_tensor.shape[0],), jnp.int32),
            pltpu.VMEM((check_tensor.shape[0],), jnp.int32),
            pltpu.VMEM((per_w,), jnp.int32),
            pltpu.VMEM((per_w,), jnp.int32),
        ],
        compiler_params=pltpu.CompilerParams(needs_layout_passes=False),
    )
    return f(x, check_tensor)
